# baseline (speedup 1.0000x reference)
import functools

import jax
import jax.numpy as jnp
from jax import lax
from jax.experimental import pallas as pl
from jax.experimental.pallas import tpu as pltpu
from jax.experimental.pallas import tpu_sc as plsc

NC = 2
NS = 16
L = 16
NW = NC * NS

B = 16384
D = 64
CHUNK = 128
B_PER_W = B // NW
NCHUNK = B_PER_W // CHUNK


def _bpr_body(u_tab, i_tab, uidx_hbm, pidx_hbm, nidx_hbm, out_hbm,
              uidx_v, pidx_v, nidx_v, urows, prows, nrows, out_v, sem):
    wid = lax.axis_index("s") * NC + lax.axis_index("c")
    chunk0 = wid * NCHUNK

    pltpu.sync_copy(uidx_hbm.at[pl.ds(chunk0, NCHUNK)], uidx_v)
    pltpu.sync_copy(pidx_hbm.at[pl.ds(chunk0, NCHUNK)], pidx_v)
    pltpu.sync_copy(nidx_hbm.at[pl.ds(chunk0, NCHUNK)], nidx_v)

    copies = []
    for j in range(NCHUNK):
        dst = pl.ds(j * CHUNK, CHUNK)
        copies.append(pltpu.async_copy(u_tab.at[uidx_v.at[j]], urows.at[dst], sem))
        copies.append(pltpu.async_copy(i_tab.at[pidx_v.at[j]], prows.at[dst], sem))
        copies.append(pltpu.async_copy(i_tab.at[nidx_v.at[j]], nrows.at[dst], sem))
    for c in copies:
        c.wait()

    iota = lax.iota(jnp.int32, L)

    def group_body(g, carry):
        rb = g * L
        rowids = rb + iota
        acc = jnp.zeros((L,), jnp.float32)
        for d in range(D):
            dv = jnp.full((L,), d, jnp.int32)
            u = plsc.load_gather(urows, [rowids, dv])
            p = plsc.load_gather(prows, [rowids, dv])
            n = plsc.load_gather(nrows, [rowids, dv])
            acc = acc + u * (p - n)
        out_v[pl.ds(rb, L)] = acc
        return carry

    lax.fori_loop(0, B_PER_W // L, group_body, 0)

    pltpu.sync_copy(out_v, out_hbm.at[pl.ds(wid * B_PER_W, B_PER_W)])


@jax.jit
def _bpr_sc(user_table, item_table, uidx, pidx, nidx):
    mesh = plsc.VectorSubcoreMesh(
        core_axis_name="c", subcore_axis_name="s", num_cores=NC, num_subcores=NS
    )
    return pl.kernel(
        _bpr_body,
        out_type=jax.ShapeDtypeStruct((B,), jnp.float32),
        mesh=mesh,
        scratch_types=[
            pltpu.VMEM((NCHUNK, CHUNK), jnp.int32),
            pltpu.VMEM((NCHUNK, CHUNK), jnp.int32),
            pltpu.VMEM((NCHUNK, CHUNK), jnp.int32),
            pltpu.VMEM((B_PER_W, D), jnp.float32),
            pltpu.VMEM((B_PER_W, D), jnp.float32),
            pltpu.VMEM((B_PER_W, D), jnp.float32),
            pltpu.VMEM((B_PER_W,), jnp.float32),
            pltpu.SemaphoreType.DMA,
        ],
        compiler_params=pltpu.CompilerParams(
            needs_layout_passes=False, use_tc_tiling_on_sc=False),
    )(user_table, item_table, uidx, pidx, nidx)


def kernel(user_table, item_table, user_input, pos_item_input, neg_item_input):
    uidx = user_input.reshape(NW * NCHUNK, CHUNK).astype(jnp.int32)
    pidx = pos_item_input.reshape(NW * NCHUNK, CHUNK).astype(jnp.int32)
    nidx = neg_item_input.reshape(NW * NCHUNK, CHUNK).astype(jnp.int32)
    out = _bpr_sc(user_table, item_table, uidx, pidx, nidx)
    return out.reshape(B, 1)
